# Initial kernel scaffold; baseline (speedup 1.0000x reference)
import jax
import jax.numpy as jnp
from jax.experimental import pallas as pl


def kernel(z, edge_label_index):
    raise NotImplementedError("write your pallas kernel here")


# speedup vs baseline: 1.9155x; 1.9155x over previous
import functools

import jax
import jax.numpy as jnp
from jax import lax
from jax.experimental import pallas as pl
from jax.experimental.pallas import tpu as pltpu
from jax.experimental.pallas import tpu_sc as plsc

L = 16
NC = 2
NS = 16
NW = NC * NS
C = 128
D = 128


@functools.partial(jax.jit, static_argnames=("n_chunks",))
def _decode(z, src, dst, n_chunks):
    e_pad = NW * C * n_chunks
    mesh = plsc.VectorSubcoreMesh(core_axis_name="c", subcore_axis_name="s")

    @functools.partial(
        pl.kernel,
        mesh=mesh,
        out_type=jax.ShapeDtypeStruct((e_pad,), jnp.float32),
        scratch_types=[
            pltpu.VMEM((C,), jnp.int32),
            pltpu.VMEM((C,), jnp.int32),
            pltpu.VMEM((C, D), jnp.float32),
            pltpu.VMEM((C, D), jnp.float32),
            pltpu.VMEM((C,), jnp.float32),
            pltpu.SemaphoreType.DMA,
        ],
    )
    def k(z_hbm, src_hbm, dst_hbm, out_hbm, sidx, didx, srows, drows, obuf, sem):
        wid = lax.axis_index("s") * NC + lax.axis_index("c")

        def chunk_body(c, carry):
            base = (wid * n_chunks + c) * C
            pltpu.sync_copy(src_hbm.at[pl.ds(base, C)], sidx)
            pltpu.sync_copy(dst_hbm.at[pl.ds(base, C)], didx)
            cp1 = pltpu.async_copy(z_hbm.at[sidx], srows, sem)
            cp2 = pltpu.async_copy(z_hbm.at[didx], drows, sem)
            cp1.wait()
            cp2.wait()

            lane = lax.iota(jnp.int32, L)
            perms = [lane ^ (1 << p) for p in range(4)]

            def group_body(g, carry2):
                out_vec = jnp.zeros((L,), jnp.float32)
                for u in range(L):
                    e = g * L + u
                    acc = jnp.zeros((L,), jnp.float32)
                    for j in range(D // L):
                        s = srows[e, pl.ds(j * L, L)]
                        t = drows[e, pl.ds(j * L, L)]
                        acc = acc + s * t
                    for p in perms:
                        acc = acc + jnp.take(acc, p)
                    out_vec = jnp.where(lane == u, acc, out_vec)
                obuf[pl.ds(g * L, L)] = out_vec
                return carry2

            lax.fori_loop(0, C // L, group_body, 0, unroll=False)
            pltpu.sync_copy(obuf, out_hbm.at[pl.ds(base, C)])
            return carry

        lax.fori_loop(0, n_chunks, chunk_body, 0, unroll=False)

    return k(z, src, dst)


def kernel(z, edge_label_index):
    e = edge_label_index.shape[1]
    idx = edge_label_index.astype(jnp.int32)
    per_round = NW * C
    n_chunks = (e + per_round - 1) // per_round
    pad = n_chunks * per_round - e
    src = jnp.pad(idx[0], (0, pad))
    dst = jnp.pad(idx[1], (0, pad))
    out = _decode(z, src, dst, n_chunks)
    return out[:e]
